# Initial kernel scaffold; baseline (speedup 1.0000x reference)
import jax
import jax.numpy as jnp
from jax.experimental import pallas as pl


def kernel(x_user, x_merchant, edge_index_user_merchant, edge_index_merchant_user, W_emb_user, b_emb_user, W_emb_merchant, b_emb_merchant, Wconv_um, bconv_um, Wconv_mu, bconv_mu, Wc1, bc1, Wc2, bc2):
    raise NotImplementedError("write your pallas kernel here")


# speedup vs baseline: 12.2703x; 12.2703x over previous
import functools

import jax
import jax.numpy as jnp
from jax import lax
from jax.experimental import pallas as pl
from jax.experimental.pallas import tpu as pltpu
from jax.experimental.pallas import tpu_sc as plsc

NU, NM, DIN, H, NL, E = 50000, 10000, 128, 64, 3, 800000
NUP, NMP = 50048, 10112
NC, NS = 2, 16
NW = NC * NS
K = 128
EP = 819200
RT = EP // (NW * K)
HH = H // 2

_f32 = jnp.float32
_MESH = plsc.VectorSubcoreMesh(core_axis_name="c", subcore_axis_name="s",
                               num_cores=NC, num_subcores=NS)


def _sds(shape):
    return jax.ShapeDtypeStruct(shape, _f32)


@functools.partial(
    pl.kernel,
    out_type=[_sds((NC * NUP,)), _sds((NC * NMP,)), _sds((NC * NMP,)), _sds((NC * NUP,))],
    mesh=_MESH,
    compiler_params=pltpu.CompilerParams(use_tc_tiling_on_sc=False),
    scratch_types=[
        pltpu.VMEM((RT, K), jnp.int32),
        pltpu.VMEM((K,), _f32),
        pltpu.VMEM((1024,), _f32),
        pltpu.VMEM((1024,), _f32),
        pltpu.VMEM_SHARED((NUP,), _f32),
        pltpu.VMEM_SHARED((NMP,), _f32),
        pltpu.VMEM_SHARED((NMP,), _f32),
        pltpu.VMEM_SHARED((NUP,), _f32),
    ],
)
def _deg_kernel(um_s, um_d, mu_s, mu_d, o_ua, o_ma, o_mb, o_ub,
                idx_v, ones_v, zb, vstage, sh_ua, sh_ma, sh_mb, sh_ub):
    c = lax.axis_index("c")
    s = lax.axis_index("s")
    wid = s * NC + c

    for i in range(8):
        ones_v[pl.ds(16 * i, 16)] = jnp.ones((16,), _f32)

    def mz(i, _):
        zb[pl.ds(i * 16, 16)] = jnp.zeros((16,), _f32)
        return 0
    lax.fori_loop(0, 64, mz, 0)

    uo = s * (NUP // NS)
    for j in range(3):
        pltpu.sync_copy(zb, sh_ua.at[pl.ds(uo + 1024 * j, 1024)])
        pltpu.sync_copy(zb, sh_ub.at[pl.ds(uo + 1024 * j, 1024)])
    pltpu.sync_copy(zb.at[pl.ds(0, 56)], sh_ua.at[pl.ds(uo + 3072, 56)])
    pltpu.sync_copy(zb.at[pl.ds(0, 56)], sh_ub.at[pl.ds(uo + 3072, 56)])
    mo = s * (NMP // NS)
    pltpu.sync_copy(zb.at[pl.ds(0, 632)], sh_ma.at[pl.ds(mo, 632)])
    pltpu.sync_copy(zb.at[pl.ds(0, 632)], sh_mb.at[pl.ds(mo, 632)])
    plsc.subcore_barrier()

    def scatter_ones(arr, sh):
        pltpu.sync_copy(arr.at[pl.ds(wid * RT, RT)], idx_v)

        def body(j, _):
            pltpu.sync_copy(ones_v, sh.at[idx_v.at[j]], add=True)
            return 0
        lax.fori_loop(0, RT, body, 0)

    scatter_ones(um_s, sh_ua)
    scatter_ones(um_d, sh_ma)
    scatter_ones(mu_s, sh_mb)
    scatter_ones(mu_d, sh_ub)
    plsc.subcore_barrier()

    def out_copy(sh, o, so, oo, sz):
        pltpu.sync_copy(sh.at[pl.ds(so, sz)], vstage.at[pl.ds(0, sz)])
        pltpu.sync_copy(vstage.at[pl.ds(0, sz)], o.at[pl.ds(oo, sz)])

    for sh, o, base in ((sh_ua, o_ua, c * NUP), (sh_ub, o_ub, c * NUP)):
        for j in range(3):
            out_copy(sh, o, uo + 1024 * j, base + uo + 1024 * j, 1024)
        out_copy(sh, o, uo + 3072, base + uo + 3072, 56)
    for sh, o, base in ((sh_ma, o_ma, c * NMP), (sh_mb, o_mb, c * NMP)):
        out_copy(sh, o, mo, base + mo, 632)


@functools.partial(
    pl.kernel,
    out_type=[_sds((NC, NMP, HH)), _sds((NC, NMP, HH)),
              _sds((NC, NUP, HH)), _sds((NC, NUP, HH))],
    mesh=_MESH,
    compiler_params=pltpu.CompilerParams(use_tc_tiling_on_sc=False),
    scratch_types=[
        pltpu.VMEM((4, K), jnp.int32),
        pltpu.VMEM((4, K), jnp.int32),
        pltpu.VMEM((K, HH), _f32),
        pltpu.VMEM((160, HH), _f32),
        pltpu.VMEM_SHARED((NMP + NUP, HH), _f32),
        pltpu.SemaphoreType.DMA,
    ],
)
def _agg_kernel(tu0, tu1, tm0, tm1, um_s, um_d, mu_s, mu_d,
                o_m0, o_m1, o_u0, o_u1,
                sidx, didx, rows_v, stage, buf, sem):
    c = lax.axis_index("c")
    s = lax.axis_index("s")
    wid = s * NC + c

    def mz(i, _):
        stage[i, pl.ds(0, 16)] = jnp.zeros((16,), _f32)
        stage[i, pl.ds(16, 16)] = jnp.zeros((16,), _f32)
        return 0
    lax.fori_loop(0, 32, mz, 0)

    def phase(tab, tab_base, nsrcp, s_arr, d_arr, acc_base, ndstp, out_ref):
        srpt = nsrcp // NS
        soff = s * srpt
        snf, stl = srpt // K, srpt % K

        def ld(j, _):
            pltpu.sync_copy(tab.at[pl.ds(soff + K * j, K)], stage.at[pl.ds(32, K)])
            pltpu.sync_copy(stage.at[pl.ds(32, K)],
                            buf.at[pl.ds(tab_base + soff + K * j, K)])
            return 0
        lax.fori_loop(0, snf, ld, 0)
        if stl:
            pltpu.sync_copy(tab.at[pl.ds(soff + K * snf, stl)],
                            stage.at[pl.ds(32, stl)])
            pltpu.sync_copy(stage.at[pl.ds(32, stl)],
                            buf.at[pl.ds(tab_base + soff + K * snf, stl)])

        rpt = ndstp // NS
        off = s * rpt
        nfull, tail = rpt // 32, rpt % 32

        def zr(j, _):
            pltpu.sync_copy(stage.at[pl.ds(0, 32)],
                            buf.at[pl.ds(acc_base + off + 32 * j, 32)])
            return 0
        lax.fori_loop(0, nfull, zr, 0)
        if tail:
            pltpu.sync_copy(stage.at[pl.ds(0, tail)],
                            buf.at[pl.ds(acc_base + off + 32 * nfull, tail)])
        plsc.subcore_barrier()

        def body(jc, _):
            pltpu.sync_copy(s_arr.at[pl.ds(wid * RT + 4 * jc, 4)], sidx)
            pltpu.sync_copy(d_arr.at[pl.ds(wid * RT + 4 * jc, 4)], didx)
            for j in range(4):
                pltpu.async_copy(buf.at[sidx.at[j]], rows_v, sem).wait()
                pltpu.sync_copy(rows_v, buf.at[didx.at[j]], add=True)
            return 0
        lax.fori_loop(0, RT // 4, body, 0)
        plsc.subcore_barrier()

        onf, otl = rpt // K, rpt % K

        def st(j, _):
            pltpu.sync_copy(buf.at[pl.ds(acc_base + off + K * j, K)],
                            stage.at[pl.ds(32, K)])
            pltpu.sync_copy(stage.at[pl.ds(32, K)], out_ref.at[c, pl.ds(off + K * j, K)])
            return 0
        lax.fori_loop(0, onf, st, 0)
        if otl:
            pltpu.sync_copy(buf.at[pl.ds(acc_base + off + K * onf, otl)],
                            stage.at[pl.ds(32, otl)])
            pltpu.sync_copy(stage.at[pl.ds(32, otl)],
                            out_ref.at[c, pl.ds(off + K * onf, otl)])
        plsc.subcore_barrier()

    phase(tu0, NMP, NUP, um_s, um_d, 0, NMP, o_m0)
    phase(tu1, NMP, NUP, um_s, um_d, 0, NMP, o_m1)
    phase(tm0, 0, NMP, mu_s, mu_d, NMP, NUP, o_u0)
    phase(tm1, 0, NMP, mu_s, mu_d, NMP, NUP, o_u1)


def _rfac_body(h1_ref, h2_ref, r1_ref, r2_ref):
    r1_ref[...] = lax.rsqrt(jnp.clip(jnp.sum(h1_ref[...], axis=0), 1.0))
    r2_ref[...] = lax.rsqrt(jnp.clip(jnp.sum(h2_ref[...], axis=0), 1.0))


def _rfac(h1, h2, n, blk):
    grid = n // blk
    return pl.pallas_call(
        _rfac_body,
        grid=(grid,),
        in_specs=[pl.BlockSpec((NC, blk, 1), lambda i: (0, i, 0))] * 2,
        out_specs=[pl.BlockSpec((blk, 1), lambda i: (i, 0))] * 2,
        out_shape=[_sds((n, 1))] * 2,
    )(h1, h2)


def _embed_body(x_ref, w_ref, b_ref, r_ref, t0_ref, t1_ref):
    h = jnp.dot(x_ref[...], w_ref[...], preferred_element_type=_f32,
                precision=lax.Precision.HIGHEST) + b_ref[...]
    h = h * r_ref[...]
    t0_ref[...] = h[:, :HH]
    t1_ref[...] = h[:, HH:]


def _embed(x, w, b, r, n, blk):
    grid = n // blk
    return pl.pallas_call(
        _embed_body,
        grid=(grid,),
        in_specs=[
            pl.BlockSpec((blk, DIN), lambda i: (i, 0)),
            pl.BlockSpec((DIN, H), lambda i: (0, 0)),
            pl.BlockSpec((1, H), lambda i: (0, 0)),
            pl.BlockSpec((blk, 1), lambda i: (i, 0)),
        ],
        out_specs=[pl.BlockSpec((blk, HH), lambda i: (i, 0))] * 2,
        out_shape=[_sds((n, HH))] * 2,
    )(x, w, b, r)


def _layer_body(a0_ref, a1_ref, rd_ref, w_ref, b_ref, rn_ref, t0_ref, t1_ref):
    a = jnp.concatenate([a0_ref[0] + a0_ref[1], a1_ref[0] + a1_ref[1]], axis=1)
    a = a * rd_ref[...]
    h = jnp.dot(a, w_ref[...], preferred_element_type=_f32,
                precision=lax.Precision.HIGHEST) + b_ref[...]
    h = jnp.maximum(h, 0.0) * rn_ref[...]
    t0_ref[...] = h[:, :HH]
    t1_ref[...] = h[:, HH:]


def _layer(a0, a1, rd, w, b, rn, n, blk):
    grid = n // blk
    return pl.pallas_call(
        _layer_body,
        grid=(grid,),
        in_specs=[
            pl.BlockSpec((NC, blk, HH), lambda i: (0, i, 0)),
            pl.BlockSpec((NC, blk, HH), lambda i: (0, i, 0)),
            pl.BlockSpec((blk, 1), lambda i: (i, 0)),
            pl.BlockSpec((H, H), lambda i: (0, 0)),
            pl.BlockSpec((1, H), lambda i: (0, 0)),
            pl.BlockSpec((blk, 1), lambda i: (i, 0)),
        ],
        out_specs=[pl.BlockSpec((blk, HH), lambda i: (i, 0))] * 2,
        out_shape=[_sds((n, HH))] * 2,
    )(a0, a1, rd, w, b, rn)


def _last_body(a0_ref, a1_ref, rd_ref, w_ref, b_ref, cs_ref, *, blk, nreal):
    i = pl.program_id(0)
    a = jnp.concatenate([a0_ref[0] + a0_ref[1], a1_ref[0] + a1_ref[1]], axis=1)
    a = a * rd_ref[...]
    h = jnp.dot(a, w_ref[...], preferred_element_type=_f32,
                precision=lax.Precision.HIGHEST) + b_ref[...]
    h = jnp.maximum(h, 0.0)
    rows = i * blk + lax.broadcasted_iota(jnp.int32, (blk, 1), 0)
    h = jnp.where(rows < nreal, h, 0.0)

    @pl.when(i == 0)
    def _():
        cs_ref[...] = jnp.zeros_like(cs_ref)
    cs_ref[...] += jnp.sum(h, axis=0, keepdims=True)


def _last(a0, a1, rd, w, b, n, blk, nreal):
    grid = n // blk
    return pl.pallas_call(
        functools.partial(_last_body, blk=blk, nreal=nreal),
        grid=(grid,),
        in_specs=[
            pl.BlockSpec((NC, blk, HH), lambda i: (0, i, 0)),
            pl.BlockSpec((NC, blk, HH), lambda i: (0, i, 0)),
            pl.BlockSpec((blk, 1), lambda i: (i, 0)),
            pl.BlockSpec((H, H), lambda i: (0, 0)),
            pl.BlockSpec((1, H), lambda i: (0, 0)),
        ],
        out_specs=pl.BlockSpec((1, H), lambda i: (0, 0)),
        out_shape=_sds((1, H)),
    )(a0, a1, rd, w, b)


def _readout_body(cu_ref, cm_ref, w1_ref, b1_ref, w2_ref, b2_ref, o_ref):
    g = cu_ref[...] / NU / 2.0 + cm_ref[...] / NM / 2.0
    z = jnp.maximum(jnp.dot(g, w1_ref[...], preferred_element_type=_f32,
                            precision=lax.Precision.HIGHEST) + b1_ref[...], 0.0)
    o = jnp.dot(z, w2_ref[...], preferred_element_type=_f32,
                precision=lax.Precision.HIGHEST) + b2_ref[...]
    o_ref[...] = 1.0 / (1.0 + jnp.exp(-o))


def _readout(cu, cm, w1, b1, w2, b2):
    return pl.pallas_call(
        _readout_body,
        out_shape=_sds((1, 1)),
    )(cu, cm, w1, b1, w2, b2)


def kernel(x_user, x_merchant, edge_index_user_merchant, edge_index_merchant_user,
           W_emb_user, b_emb_user, W_emb_merchant, b_emb_merchant,
           Wconv_um, bconv_um, Wconv_mu, bconv_mu, Wc1, bc1, Wc2, bc2):
    def pad_idx(a, nreal, npad, shift):
        a = a.astype(jnp.int32)
        pad = nreal + jnp.arange(EP - E, dtype=jnp.int32) % (npad - nreal)
        return (jnp.concatenate([a, pad]) + shift).reshape(EP // K, K)

    um_s = pad_idx(edge_index_user_merchant[0], NU, NUP, 0)
    um_d = pad_idx(edge_index_user_merchant[1], NM, NMP, 0)
    mu_s = pad_idx(edge_index_merchant_user[0], NM, NMP, 0)
    mu_d = pad_idx(edge_index_merchant_user[1], NU, NUP, 0)
    um_s_sh = pad_idx(edge_index_user_merchant[0], NU, NUP, NMP)
    mu_d_sh = pad_idx(edge_index_merchant_user[1], NU, NUP, NMP)

    xu = jnp.zeros((NUP, DIN), _f32).at[:NU].set(x_user)
    xm = jnp.zeros((NMP, DIN), _f32).at[:NM].set(x_merchant)

    h_ua, h_ma, h_mb, h_ub = _deg_kernel(um_s, um_d, mu_s, mu_d)
    r_um_src, r_mu_dst = _rfac(h_ua.reshape(NC, NUP, 1), h_ub.reshape(NC, NUP, 1),
                               NUP, 6256)
    r_mu_src, r_um_dst = _rfac(h_mb.reshape(NC, NMP, 1), h_ma.reshape(NC, NMP, 1),
                               NMP, 2528)

    tu0, tu1 = _embed(xu, W_emb_user, b_emb_user.reshape(1, H), r_um_src, NUP, 3128)
    tm0, tm1 = _embed(xm, W_emb_merchant, b_emb_merchant.reshape(1, H), r_mu_src,
                      NMP, 2528)

    for l in range(NL):
        o_m0, o_m1, o_u0, o_u1 = _agg_kernel(tu0, tu1, tm0, tm1,
                                             um_s_sh, um_d, mu_s, mu_d_sh)
        wm, bm = Wconv_um[l], bconv_um[l].reshape(1, H)
        wu, bu = Wconv_mu[l], bconv_mu[l].reshape(1, H)
        if l < NL - 1:
            tu0, tu1 = _layer(o_u0, o_u1, r_mu_dst, wu, bu, r_um_src, NUP, 3128)
            tm0, tm1 = _layer(o_m0, o_m1, r_um_dst, wm, bm, r_mu_src, NMP, 2528)
        else:
            cs_u = _last(o_u0, o_u1, r_mu_dst, wu, bu, NUP, 3128, NU)
            cs_m = _last(o_m0, o_m1, r_um_dst, wm, bm, NMP, 2528, NM)

    out = _readout(cs_u, cs_m, Wc1, bc1.reshape(1, H), Wc2, bc2.reshape(1, 1))
    return out.reshape(1)


# speedup vs baseline: 17.1188x; 1.3951x over previous
import functools

import jax
import jax.numpy as jnp
from jax import lax
from jax.experimental import pallas as pl
from jax.experimental.pallas import tpu as pltpu
from jax.experimental.pallas import tpu_sc as plsc

NU, NM, DIN, H, NL, E = 50000, 10000, 128, 64, 3, 800000
NUP, NMP = 50048, 10112
NC, NS = 2, 16
NW = NC * NS
K = 128
EP = 819200
RT = EP // (NW * K)
HH = H // 2

_f32 = jnp.float32
_MESH = plsc.VectorSubcoreMesh(core_axis_name="c", subcore_axis_name="s",
                               num_cores=NC, num_subcores=NS)


def _sds(shape):
    return jax.ShapeDtypeStruct(shape, _f32)


@functools.partial(
    pl.kernel,
    out_type=[_sds((NC * NUP,)), _sds((NC * NMP,)), _sds((NC * NMP,)), _sds((NC * NUP,))],
    mesh=_MESH,
    compiler_params=pltpu.CompilerParams(use_tc_tiling_on_sc=False),
    scratch_types=[
        pltpu.VMEM((RT, K), jnp.int32),
        pltpu.VMEM((K,), _f32),
        pltpu.VMEM((1024,), _f32),
        pltpu.VMEM((1024,), _f32),
        pltpu.VMEM_SHARED((NUP,), _f32),
        pltpu.VMEM_SHARED((NMP,), _f32),
        pltpu.VMEM_SHARED((NMP,), _f32),
        pltpu.VMEM_SHARED((NUP,), _f32),
    ],
)
def _deg_kernel(um_s, um_d, mu_s, mu_d, o_ua, o_ma, o_mb, o_ub,
                idx_v, ones_v, zb, vstage, sh_ua, sh_ma, sh_mb, sh_ub):
    c = lax.axis_index("c")
    s = lax.axis_index("s")
    wid = s * NC + c

    for i in range(8):
        ones_v[pl.ds(16 * i, 16)] = jnp.ones((16,), _f32)

    def mz(i, _):
        zb[pl.ds(i * 16, 16)] = jnp.zeros((16,), _f32)
        return 0
    lax.fori_loop(0, 64, mz, 0)

    uo = s * (NUP // NS)
    for j in range(3):
        pltpu.sync_copy(zb, sh_ua.at[pl.ds(uo + 1024 * j, 1024)])
        pltpu.sync_copy(zb, sh_ub.at[pl.ds(uo + 1024 * j, 1024)])
    pltpu.sync_copy(zb.at[pl.ds(0, 56)], sh_ua.at[pl.ds(uo + 3072, 56)])
    pltpu.sync_copy(zb.at[pl.ds(0, 56)], sh_ub.at[pl.ds(uo + 3072, 56)])
    mo = s * (NMP // NS)
    pltpu.sync_copy(zb.at[pl.ds(0, 632)], sh_ma.at[pl.ds(mo, 632)])
    pltpu.sync_copy(zb.at[pl.ds(0, 632)], sh_mb.at[pl.ds(mo, 632)])
    plsc.subcore_barrier()

    def scatter_ones(arr, sh):
        pltpu.sync_copy(arr.at[pl.ds(wid * RT, RT)], idx_v)

        def body(j, _):
            pltpu.sync_copy(ones_v, sh.at[idx_v.at[j]], add=True)
            return 0
        lax.fori_loop(0, RT, body, 0)

    scatter_ones(um_s, sh_ua)
    scatter_ones(um_d, sh_ma)
    scatter_ones(mu_s, sh_mb)
    scatter_ones(mu_d, sh_ub)
    plsc.subcore_barrier()

    def out_copy(sh, o, so, oo, sz):
        pltpu.sync_copy(sh.at[pl.ds(so, sz)], vstage.at[pl.ds(0, sz)])
        pltpu.sync_copy(vstage.at[pl.ds(0, sz)], o.at[pl.ds(oo, sz)])

    for sh, o, base in ((sh_ua, o_ua, c * NUP), (sh_ub, o_ub, c * NUP)):
        for j in range(3):
            out_copy(sh, o, uo + 1024 * j, base + uo + 1024 * j, 1024)
        out_copy(sh, o, uo + 3072, base + uo + 3072, 56)
    for sh, o, base in ((sh_ma, o_ma, c * NMP), (sh_mb, o_mb, c * NMP)):
        out_copy(sh, o, mo, base + mo, 632)


@functools.partial(
    pl.kernel,
    out_type=[_sds((NC, NMP, HH)), _sds((NC, NMP, HH)),
              _sds((NC, NUP, HH)), _sds((NC, NUP, HH))],
    mesh=_MESH,
    compiler_params=pltpu.CompilerParams(use_tc_tiling_on_sc=False),
    scratch_types=[
        pltpu.VMEM((4, 2, K), jnp.int32),
        pltpu.VMEM((2, K, HH), _f32),
        pltpu.VMEM((32, HH), _f32),
        pltpu.VMEM_SHARED((NMP + NUP, HH), _f32),
        pltpu.SemaphoreType.DMA,
        pltpu.SemaphoreType.DMA,
        pltpu.SemaphoreType.DMA,
    ],
)
def _agg_kernel(tu0, tu1, tm0, tm1, um_sd, mu_sd,
                o_m0, o_m1, o_u0, o_u1,
                sd_v, rows2, zb, buf, sem_i, sem_g, sem_s):
    c = lax.axis_index("c")
    s = lax.axis_index("s")
    wid = s * NC + c

    def mz(i, _):
        zb[i, pl.ds(0, 16)] = jnp.zeros((16,), _f32)
        zb[i, pl.ds(16, 16)] = jnp.zeros((16,), _f32)
        return 0
    lax.fori_loop(0, 32, mz, 0)

    def phase(tab, tab_base, nsrcp, sd_arr, acc_base, ndstp, out_ref):
        srpt = nsrcp // NS
        soff = s * srpt
        snf, stl = srpt // K, srpt % K

        def ld(j, _):
            pltpu.sync_copy(tab.at[pl.ds(soff + K * j, K)], rows2.at[0])
            pltpu.sync_copy(rows2.at[0], buf.at[pl.ds(tab_base + soff + K * j, K)])
            return 0
        lax.fori_loop(0, snf, ld, 0)
        if stl:
            pltpu.sync_copy(tab.at[pl.ds(soff + K * snf, stl)],
                            rows2.at[0, pl.ds(0, stl)])
            pltpu.sync_copy(rows2.at[0, pl.ds(0, stl)],
                            buf.at[pl.ds(tab_base + soff + K * snf, stl)])

        rpt = ndstp // NS
        off = s * rpt
        nfull, tail = rpt // 32, rpt % 32

        def zr(j, _):
            pltpu.sync_copy(zb, buf.at[pl.ds(acc_base + off + 32 * j, 32)])
            return 0
        lax.fori_loop(0, nfull, zr, 0)
        if tail:
            pltpu.sync_copy(zb.at[pl.ds(0, tail)],
                            buf.at[pl.ds(acc_base + off + 32 * nfull, tail)])
        plsc.subcore_barrier()

        base = wid * RT
        pltpu.async_copy(sd_arr.at[base], sd_v.at[0], sem_i)
        pltpu.async_copy(sd_arr.at[base + 1], sd_v.at[1], sem_i)

        def body(j, _):
            slot = lax.rem(j, 4)
            b = lax.rem(j, 2)

            @pl.when(j >= 2)
            def _():
                pltpu.make_async_copy(tab.at[pl.ds(0, K)], rows2.at[0], sem_s).wait()

            pltpu.make_async_copy(sd_arr.at[base], sd_v.at[0], sem_i).wait()
            g = pltpu.async_copy(buf.at[sd_v.at[slot, 0]], rows2.at[b], sem_g)

            @pl.when(j + 2 < RT)
            def _():
                pltpu.async_copy(sd_arr.at[base + j + 2],
                                 sd_v.at[lax.rem(j + 2, 4)], sem_i)
            g.wait()
            pltpu.async_copy(rows2.at[b], buf.at[sd_v.at[slot, 1]], sem_s, add=True)
            return 0
        lax.fori_loop(0, RT, body, 0)
        pltpu.make_async_copy(tab.at[pl.ds(0, K)], rows2.at[0], sem_s).wait()
        pltpu.make_async_copy(tab.at[pl.ds(0, K)], rows2.at[0], sem_s).wait()
        plsc.subcore_barrier()

        onf, otl = rpt // K, rpt % K

        def st(j, _):
            pltpu.sync_copy(buf.at[pl.ds(acc_base + off + K * j, K)], rows2.at[0])
            pltpu.sync_copy(rows2.at[0], out_ref.at[c, pl.ds(off + K * j, K)])
            return 0
        lax.fori_loop(0, onf, st, 0)
        if otl:
            pltpu.sync_copy(buf.at[pl.ds(acc_base + off + K * onf, otl)],
                            rows2.at[0, pl.ds(0, otl)])
            pltpu.sync_copy(rows2.at[0, pl.ds(0, otl)],
                            out_ref.at[c, pl.ds(off + K * onf, otl)])
        plsc.subcore_barrier()

    phase(tu0, NMP, NUP, um_sd, 0, NMP, o_m0)
    phase(tu1, NMP, NUP, um_sd, 0, NMP, o_m1)
    phase(tm0, 0, NMP, mu_sd, NMP, NUP, o_u0)
    phase(tm1, 0, NMP, mu_sd, NMP, NUP, o_u1)


def _rfac_body(h1_ref, h2_ref, r1_ref, r2_ref):
    r1_ref[...] = lax.rsqrt(jnp.clip(jnp.sum(h1_ref[...], axis=0), 1.0))
    r2_ref[...] = lax.rsqrt(jnp.clip(jnp.sum(h2_ref[...], axis=0), 1.0))


def _rfac(h1, h2, n, blk):
    grid = n // blk
    return pl.pallas_call(
        _rfac_body,
        grid=(grid,),
        in_specs=[pl.BlockSpec((NC, blk, 1), lambda i: (0, i, 0))] * 2,
        out_specs=[pl.BlockSpec((blk, 1), lambda i: (i, 0))] * 2,
        out_shape=[_sds((n, 1))] * 2,
    )(h1, h2)


def _embed_body(x_ref, w_ref, b_ref, r_ref, t0_ref, t1_ref):
    h = jnp.dot(x_ref[...], w_ref[...], preferred_element_type=_f32,
                precision=lax.Precision.HIGHEST) + b_ref[...]
    h = h * r_ref[...]
    t0_ref[...] = h[:, :HH]
    t1_ref[...] = h[:, HH:]


def _embed(x, w, b, r, n, blk):
    grid = n // blk
    return pl.pallas_call(
        _embed_body,
        grid=(grid,),
        in_specs=[
            pl.BlockSpec((blk, DIN), lambda i: (i, 0)),
            pl.BlockSpec((DIN, H), lambda i: (0, 0)),
            pl.BlockSpec((1, H), lambda i: (0, 0)),
            pl.BlockSpec((blk, 1), lambda i: (i, 0)),
        ],
        out_specs=[pl.BlockSpec((blk, HH), lambda i: (i, 0))] * 2,
        out_shape=[_sds((n, HH))] * 2,
    )(x, w, b, r)


def _layer_body(a0_ref, a1_ref, rd_ref, w_ref, b_ref, rn_ref, t0_ref, t1_ref):
    a = jnp.concatenate([a0_ref[0] + a0_ref[1], a1_ref[0] + a1_ref[1]], axis=1)
    a = a * rd_ref[...]
    h = jnp.dot(a, w_ref[...], preferred_element_type=_f32,
                precision=lax.Precision.HIGHEST) + b_ref[...]
    h = jnp.maximum(h, 0.0) * rn_ref[...]
    t0_ref[...] = h[:, :HH]
    t1_ref[...] = h[:, HH:]


def _layer(a0, a1, rd, w, b, rn, n, blk):
    grid = n // blk
    return pl.pallas_call(
        _layer_body,
        grid=(grid,),
        in_specs=[
            pl.BlockSpec((NC, blk, HH), lambda i: (0, i, 0)),
            pl.BlockSpec((NC, blk, HH), lambda i: (0, i, 0)),
            pl.BlockSpec((blk, 1), lambda i: (i, 0)),
            pl.BlockSpec((H, H), lambda i: (0, 0)),
            pl.BlockSpec((1, H), lambda i: (0, 0)),
            pl.BlockSpec((blk, 1), lambda i: (i, 0)),
        ],
        out_specs=[pl.BlockSpec((blk, HH), lambda i: (i, 0))] * 2,
        out_shape=[_sds((n, HH))] * 2,
    )(a0, a1, rd, w, b, rn)


def _last_body(a0_ref, a1_ref, rd_ref, w_ref, b_ref, cs_ref, *, blk, nreal):
    i = pl.program_id(0)
    a = jnp.concatenate([a0_ref[0] + a0_ref[1], a1_ref[0] + a1_ref[1]], axis=1)
    a = a * rd_ref[...]
    h = jnp.dot(a, w_ref[...], preferred_element_type=_f32,
                precision=lax.Precision.HIGHEST) + b_ref[...]
    h = jnp.maximum(h, 0.0)
    rows = i * blk + lax.broadcasted_iota(jnp.int32, (blk, 1), 0)
    h = jnp.where(rows < nreal, h, 0.0)

    @pl.when(i == 0)
    def _():
        cs_ref[...] = jnp.zeros_like(cs_ref)
    cs_ref[...] += jnp.sum(h, axis=0, keepdims=True)


def _last(a0, a1, rd, w, b, n, blk, nreal):
    grid = n // blk
    return pl.pallas_call(
        functools.partial(_last_body, blk=blk, nreal=nreal),
        grid=(grid,),
        in_specs=[
            pl.BlockSpec((NC, blk, HH), lambda i: (0, i, 0)),
            pl.BlockSpec((NC, blk, HH), lambda i: (0, i, 0)),
            pl.BlockSpec((blk, 1), lambda i: (i, 0)),
            pl.BlockSpec((H, H), lambda i: (0, 0)),
            pl.BlockSpec((1, H), lambda i: (0, 0)),
        ],
        out_specs=pl.BlockSpec((1, H), lambda i: (0, 0)),
        out_shape=_sds((1, H)),
    )(a0, a1, rd, w, b)


def _readout_body(cu_ref, cm_ref, w1_ref, b1_ref, w2_ref, b2_ref, o_ref):
    g = cu_ref[...] / NU / 2.0 + cm_ref[...] / NM / 2.0
    z = jnp.maximum(jnp.dot(g, w1_ref[...], preferred_element_type=_f32,
                            precision=lax.Precision.HIGHEST) + b1_ref[...], 0.0)
    o = jnp.dot(z, w2_ref[...], preferred_element_type=_f32,
                precision=lax.Precision.HIGHEST) + b2_ref[...]
    o_ref[...] = 1.0 / (1.0 + jnp.exp(-o))


def _readout(cu, cm, w1, b1, w2, b2):
    return pl.pallas_call(
        _readout_body,
        out_shape=_sds((1, 1)),
    )(cu, cm, w1, b1, w2, b2)


def kernel(x_user, x_merchant, edge_index_user_merchant, edge_index_merchant_user,
           W_emb_user, b_emb_user, W_emb_merchant, b_emb_merchant,
           Wconv_um, bconv_um, Wconv_mu, bconv_mu, Wc1, bc1, Wc2, bc2):
    def pad_idx(a, nreal, npad, shift):
        a = a.astype(jnp.int32)
        pad = nreal + jnp.arange(EP - E, dtype=jnp.int32) % (npad - nreal)
        return (jnp.concatenate([a, pad]) + shift).reshape(EP // K, K)

    um_s = pad_idx(edge_index_user_merchant[0], NU, NUP, 0)
    um_d = pad_idx(edge_index_user_merchant[1], NM, NMP, 0)
    mu_s = pad_idx(edge_index_merchant_user[0], NM, NMP, 0)
    mu_d = pad_idx(edge_index_merchant_user[1], NU, NUP, 0)
    um_sd = jnp.stack([pad_idx(edge_index_user_merchant[0], NU, NUP, NMP),
                       um_d], axis=1)
    mu_sd = jnp.stack([mu_s,
                       pad_idx(edge_index_merchant_user[1], NU, NUP, NMP)], axis=1)

    xu = jnp.zeros((NUP, DIN), _f32).at[:NU].set(x_user)
    xm = jnp.zeros((NMP, DIN), _f32).at[:NM].set(x_merchant)

    h_ua, h_ma, h_mb, h_ub = _deg_kernel(um_s, um_d, mu_s, mu_d)
    r_um_src, r_mu_dst = _rfac(h_ua.reshape(NC, NUP, 1), h_ub.reshape(NC, NUP, 1),
                               NUP, 6256)
    r_mu_src, r_um_dst = _rfac(h_mb.reshape(NC, NMP, 1), h_ma.reshape(NC, NMP, 1),
                               NMP, 2528)

    tu0, tu1 = _embed(xu, W_emb_user, b_emb_user.reshape(1, H), r_um_src, NUP, 3128)
    tm0, tm1 = _embed(xm, W_emb_merchant, b_emb_merchant.reshape(1, H), r_mu_src,
                      NMP, 2528)

    for l in range(NL):
        o_m0, o_m1, o_u0, o_u1 = _agg_kernel(tu0, tu1, tm0, tm1, um_sd, mu_sd)
        wm, bm = Wconv_um[l], bconv_um[l].reshape(1, H)
        wu, bu = Wconv_mu[l], bconv_mu[l].reshape(1, H)
        if l < NL - 1:
            tu0, tu1 = _layer(o_u0, o_u1, r_mu_dst, wu, bu, r_um_src, NUP, 3128)
            tm0, tm1 = _layer(o_m0, o_m1, r_um_dst, wm, bm, r_mu_src, NMP, 2528)
        else:
            cs_u = _last(o_u0, o_u1, r_mu_dst, wu, bu, NUP, 3128, NU)
            cs_m = _last(o_m0, o_m1, r_um_dst, wm, bm, NMP, 2528, NM)

    out = _readout(cs_u, cs_m, Wc1, bc1.reshape(1, H), Wc2, bc2.reshape(1, 1))
    return out.reshape(1)
